# baseline (speedup 1.0000x reference)
import numpy as np

import jax
import jax.numpy as jnp
from jax import lax
from jax.experimental import pallas as pl

_TS = 256
_D = 1024

_dims = np.arange(_D)
_inv_freq64 = 1.0 / (10000.0 ** (2 * (_dims // 2) / _D))
_INV_FREQ = np.asarray(_inv_freq64[None, :], dtype=np.float32)
_PARITY = np.asarray((_dims % 2)[None, :], dtype=np.int32)

_C1 = float(int(np.pi / 2 * 2**11) / 2**11)
_C2 = float(np.float32(int((np.pi / 2 - _C1) * 2**26) / 2**26))
_C3 = float(np.float32(np.pi / 2 - _C1 - _C2))
_TWO_OVER_PI = float(np.float32(2.0 / np.pi))

_S3, _S5, _S7 = -1.6666654611e-1, 8.3321608736e-3, -1.9515295891e-4
_C4, _C6, _C8 = 4.166664568298827e-2, -1.388731625493765e-3, 2.443315711809948e-5


def _body(x_ref, if_ref, par_ref, o_ref):
    i = pl.program_id(0)
    ti = (i * _TS) + lax.broadcasted_iota(jnp.int32, (_TS, _D), 0)
    arg = ti.astype(jnp.float32) * if_ref[...]
    k = (arg * _TWO_OVER_PI + 0.5).astype(jnp.int32)
    kf = k.astype(jnp.float32)
    r = arg - kf * _C1
    r = r - kf * _C2
    r = r - kf * _C3
    r2 = r * r
    sinp = ((_S7 * r2 + _S5) * r2 + _S3) * (r2 * r) + r
    cosp = (((_C8 * r2 + _C6) * r2 + _C4) * r2 - 0.5) * r2 + 1.0
    ke = k + par_ref[...]
    emb = jnp.where((ke & 1) != 0, cosp, sinp)
    emb = jnp.where((ke & 2) != 0, -emb, emb)
    o_ref[...] = x_ref[...] + emb[None, :, :]


def kernel(x, embeddings):
    B, T, D = x.shape
    return pl.pallas_call(
        _body,
        grid=(T // _TS,),
        in_specs=[
            pl.BlockSpec((B, _TS, D), lambda i: (0, i, 0)),
            pl.BlockSpec((1, D), lambda i: (0, 0)),
            pl.BlockSpec((1, D), lambda i: (0, 0)),
        ],
        out_specs=pl.BlockSpec((B, _TS, D), lambda i: (0, i, 0)),
        out_shape=jax.ShapeDtypeStruct(x.shape, x.dtype),
    )(x, _INV_FREQ, _PARITY)


# speedup vs baseline: 1.0386x; 1.0386x over previous
import numpy as np

import jax
import jax.numpy as jnp
from jax import lax
from jax.experimental import pallas as pl

_TS = 512
_D = 1024

_dims = np.arange(_D)
_inv_freq64 = 1.0 / (10000.0 ** (2 * (_dims // 2) / _D))
_INV_FREQ = np.asarray(_inv_freq64[None, :], dtype=np.float32)
_PARITY = np.asarray((_dims % 2)[None, :], dtype=np.int32)

_C1 = float(int(np.pi / 2 * 2**11) / 2**11)
_C2 = float(np.float32(int((np.pi / 2 - _C1) * 2**26) / 2**26))
_C3 = float(np.float32(np.pi / 2 - _C1 - _C2))
_TWO_OVER_PI = float(np.float32(2.0 / np.pi))

_S3, _S5, _S7 = -1.6666654611e-1, 8.3321608736e-3, -1.9515295891e-4
_C4, _C6, _C8 = 4.166664568298827e-2, -1.388731625493765e-3, 2.443315711809948e-5


def _body(x_ref, if_ref, par_ref, o_ref):
    i = pl.program_id(0)
    ti = (i * _TS) + lax.broadcasted_iota(jnp.int32, (_TS, _D), 0)
    arg = ti.astype(jnp.float32) * if_ref[...]
    k = (arg * _TWO_OVER_PI + 0.5).astype(jnp.int32)
    kf = k.astype(jnp.float32)
    r = arg - kf * _C1
    r = r - kf * _C2
    r = r - kf * _C3
    r2 = r * r
    sinp = ((_S7 * r2 + _S5) * r2 + _S3) * (r2 * r) + r
    cosp = (((_C8 * r2 + _C6) * r2 + _C4) * r2 - 0.5) * r2 + 1.0
    ke = k + par_ref[...]
    emb = jnp.where((ke & 1) != 0, cosp, sinp)
    sign = (ke & 2) << 30
    emb = lax.bitcast_convert_type(
        lax.bitcast_convert_type(emb, jnp.int32) ^ sign, jnp.float32)
    o_ref[...] = x_ref[...] + emb[None, :, :]


def kernel(x, embeddings):
    B, T, D = x.shape
    return pl.pallas_call(
        _body,
        grid=(T // _TS,),
        in_specs=[
            pl.BlockSpec((B, _TS, D), lambda i: (0, i, 0)),
            pl.BlockSpec((1, D), lambda i: (0, 0)),
            pl.BlockSpec((1, D), lambda i: (0, 0)),
        ],
        out_specs=pl.BlockSpec((B, _TS, D), lambda i: (0, i, 0)),
        out_shape=jax.ShapeDtypeStruct(x.shape, x.dtype),
    )(x, _INV_FREQ, _PARITY)
